# baseline (speedup 1.0000x reference)
import functools

import jax
import jax.numpy as jnp
from jax import lax
from jax.experimental import pallas as pl
from jax.experimental.pallas import tpu as pltpu
from jax.experimental.pallas import tpu_sc as plsc

NC = 2
NS = 16
L = 16

B = 4
T = 4096
D = 2048
K = T // 2
RADIX = 256
PASSES = 4
CHUNK = T // L

RPC = 2 * K
RPW = RPC // NS
GCH = 8
NCH = RPW // GCH


def _digit(k_i32, shift):
    ku = plsc.bitcast(k_i32, jnp.uint32)
    du = jnp.bitwise_and(jnp.right_shift(ku, jnp.uint32(shift)), jnp.uint32(RADIX - 1))
    return plsc.bitcast(du, jnp.int32)


def _desc_key(bits_i32):
    sign = jnp.right_shift(bits_i32, 31)
    mask = jnp.bitwise_and(jnp.bitwise_not(sign), jnp.int32(0x7FFFFFFF))
    return jnp.bitwise_xor(bits_i32, mask)


def _body(scores_hbm, hid_hbm, sel_hbm, bidx_hbm, idx_hbm, vals_hbm,
          sc_v, key_a, key_b, val_a, val_b, hist, offs, rowbase,
          gstage, idx_v, buf0, buf1, sh_grow, sem0, sem1):
    c = lax.axis_index("c")
    s = lax.axis_index("s")

    @pl.when(s < 2)
    def _():
        b = c * 2 + s
        pltpu.sync_copy(scores_hbm.at[pl.ds(b * T, T)], sc_v)

        lane = lax.iota(jnp.int32, L)
        lane_c = lane * CHUNK
        ones = jnp.broadcast_to(jnp.int32(1), (L,))
        fifteen = jnp.broadcast_to(jnp.int32(15), (L,))

        def phys(e):
            return e + jnp.right_shift(e, 8)

        def init_body(i, _):
            x = sc_v[pl.ds(i * L, L)]
            bits = plsc.bitcast(x, jnp.int32)
            e = lane + i * L
            pe = phys(e)
            plsc.store_scatter(key_a, [pe], _desc_key(bits))
            plsc.store_scatter(val_a, [pe], e)
            return 0
        lax.fori_loop(0, T // L, init_body, 0)

        bufs = [(key_a, val_a), (key_b, val_b)]
        for p in range(PASSES):
            shift = 8 * p
            src_k, src_v = bufs[p % 2]
            dst_k, dst_v = bufs[(p + 1) % 2]

            def zero_body(j, _):
                hist[j, :] = jnp.broadcast_to(jnp.int32(0), (L,))
                return 0
            lax.fori_loop(0, RADIX, zero_body, 0)

            def hist_body(i, _):
                idxv = lane_c + i + lane
                k = plsc.load_gather(src_k, [idxv])
                d = _digit(k, shift)
                plsc.addupdate_scatter(hist, [d, lane], ones)
                return 0
            lax.fori_loop(0, CHUNK, hist_body, 0)

            def cs1_body(dd, _):
                offs[dd, :] = plsc.cumsum(hist[dd, :])
                return 0
            lax.fori_loop(0, RADIX, cs1_body, 0)

            def rb_body(jj, carry):
                rows_vec = lane + jj * L
                tot = plsc.load_gather(offs, [rows_vec, fifteen])
                cs2 = plsc.cumsum(tot)
                plsc.store_scatter(rowbase, [rows_vec], cs2 - tot + carry)
                return carry + jnp.sum(tot)
            lax.fori_loop(0, RADIX // L, rb_body, jnp.int32(0))

            def fin_body(dd, _):
                rb_b = plsc.load_gather(rowbase, [jnp.broadcast_to(dd, (L,))])
                offs[dd, :] = offs[dd, :] - hist[dd, :] + rb_b
                return 0
            lax.fori_loop(0, RADIX, fin_body, 0)

            def perm_body(i, _):
                idxv = lane_c + i + lane
                k = plsc.load_gather(src_k, [idxv])
                v = plsc.load_gather(src_v, [idxv])
                d = _digit(k, shift)
                ofs = plsc.load_gather(offs, [d, lane])
                po = phys(ofs)
                plsc.store_scatter(dst_k, [po], k)
                plsc.store_scatter(dst_v, [po], v)
                plsc.addupdate_scatter(offs, [d, lane], ones)
                return 0
            lax.fori_loop(0, CHUNK, perm_body, 0)

        def out_body(i, _):
            pe = phys(lane + i * L)
            k = plsc.load_gather(key_a, [pe])
            v = plsc.load_gather(val_a, [pe])
            sc_v[pl.ds(i * L, L)] = plsc.bitcast(_desc_key(k), jnp.float32)
            gstage[pl.ds(i * L, L)] = v
            return 0
        lax.fori_loop(0, K // L, out_body, 0)

        pltpu.sync_copy(sc_v.at[pl.ds(0, K)], vals_hbm.at[pl.ds(b * K, K)])
        pltpu.sync_copy(gstage, idx_hbm.at[pl.ds(b * K, K)])

        def grow_body(i, _):
            gstage[pl.ds(i * L, L)] = gstage[pl.ds(i * L, L)] + b * T
            return 0
        lax.fori_loop(0, K // L, grow_body, 0)
        pltpu.sync_copy(gstage, sh_grow.at[pl.ds(s * K, K)])

        def bidx_body(i, _):
            gstage[pl.ds(i * L, L)] = jnp.broadcast_to(b, (L,))
            return 0
        lax.fori_loop(0, K // L, bidx_body, 0)
        pltpu.sync_copy(gstage, bidx_hbm.at[pl.ds(b * K, K)])

    plsc.subcore_barrier()

    pltpu.sync_copy(sh_grow.at[pl.ds(s * RPW, RPW)], idx_v)
    gbase = c * RPC + s * RPW

    bufs2 = (buf0, buf1)
    sems2 = (sem0, sem1)

    def start(ch):
        return pltpu.async_copy(
            hid_hbm.at[idx_v.at[pl.ds(ch * GCH, GCH)]], bufs2[ch % 2], sems2[ch % 2])

    pending = start(0)
    for ch in range(NCH):
        nxt = start(ch + 1) if ch + 1 < NCH else None
        pending.wait()
        pltpu.sync_copy(bufs2[ch % 2], sel_hbm.at[pl.ds(gbase + ch * GCH, GCH)])
        pending = nxt


_fused_call = functools.partial(
    pl.kernel,
    out_type=(
        jax.ShapeDtypeStruct((B * K, D), jnp.float32),
        jax.ShapeDtypeStruct((B * K,), jnp.int32),
        jax.ShapeDtypeStruct((B * K,), jnp.int32),
        jax.ShapeDtypeStruct((B * K,), jnp.float32),
    ),
    mesh=plsc.VectorSubcoreMesh(core_axis_name="c", subcore_axis_name="s"),
    compiler_params=pltpu.CompilerParams(needs_layout_passes=False),
    scratch_types=[
        pltpu.VMEM((T,), jnp.float32),
        pltpu.VMEM((T + 16,), jnp.int32),
        pltpu.VMEM((T + 16,), jnp.int32),
        pltpu.VMEM((T + 16,), jnp.int32),
        pltpu.VMEM((T + 16,), jnp.int32),
        pltpu.VMEM((RADIX, L), jnp.int32),
        pltpu.VMEM((RADIX, L), jnp.int32),
        pltpu.VMEM((RADIX,), jnp.int32),
        pltpu.VMEM((K,), jnp.int32),
        pltpu.VMEM((RPW,), jnp.int32),
        pltpu.VMEM((GCH, D), jnp.float32),
        pltpu.VMEM((GCH, D), jnp.float32),
        pltpu.VMEM_SHARED((RPC,), jnp.int32),
        pltpu.SemaphoreType.DMA,
        pltpu.SemaphoreType.DMA,
    ],
)(_body)


def kernel(scores, hidden_states):
    b, t, d = hidden_states.shape
    sel, bidx, idx, vals = _fused_call(
        scores.reshape(-1), hidden_states.reshape(b * t, d))
    return sel, bidx, idx, vals
